# baseline (speedup 1.0000x reference)
import functools

import jax
import jax.numpy as jnp
from jax import lax
from jax.experimental import pallas as pl
from jax.experimental.pallas import tpu as pltpu
from jax.experimental.pallas import tpu_sc as plsc

_VOCAB = 1000000
_EMB = 64
_B = 16384
_L = 50
_PROJ = 128

_RV = 8000
_NV = _VOCAB // _RV


def _proj_body(t_ref, wp_ref, bp_ref, o_ref):
    o_ref[...] = jnp.maximum(
        jnp.dot(t_ref[...], wp_ref[...], preferred_element_type=jnp.float32)
        + bp_ref[...], 0.0)


def _project_table(table, wp_s, bp_s):
    return pl.pallas_call(
        _proj_body,
        grid=(_NV,),
        in_specs=[
            pl.BlockSpec((_RV, _EMB), lambda i: (i, 0)),
            pl.BlockSpec((_EMB, _PROJ), lambda i: (0, 0)),
            pl.BlockSpec((1, _PROJ), lambda i: (0, 0)),
        ],
        out_specs=pl.BlockSpec((_RV, _PROJ), lambda i: (i, 0)),
        out_shape=jax.ShapeDtypeStruct((_VOCAB, _PROJ), jnp.float32),
    )(table, wp_s, bp_s)


_NC = 2
_NS = 16
_NW = _NC * _NS
_NPOOL = 2 * _B
_POOL_PER_W = _NPOOL // _NW
_CHP = 8
_CHTOK = _CHP * _L
_G = 80
_NG = _CHTOK // _G
_NCHUNK = _POOL_PER_W // _CHP
_FLUSH = 8


def _sc_pool(idx_all, ptab):
    mesh = plsc.VectorSubcoreMesh(core_axis_name="c", subcore_axis_name="s")

    @functools.partial(
        pl.kernel,
        mesh=mesh,
        out_type=jax.ShapeDtypeStruct((_NPOOL, _PROJ), jnp.float32),
        scratch_types=[
            pltpu.VMEM((_CHTOK,), jnp.int32),
            pltpu.VMEM((_CHTOK, _PROJ), jnp.float32),
            pltpu.VMEM((_FLUSH * _CHP, _PROJ), jnp.float32),
            pltpu.SemaphoreType.DMA,
        ],
    )
    def pool_kernel(idx_hbm, ptab_hbm, out_hbm, idx_v, rows_v, out_v, sem):
        wid = lax.axis_index("s") * _NC + lax.axis_index("c")
        tok_base = wid * _POOL_PER_W * _L
        row_base = wid * _POOL_PER_W

        def chunk(ci, carry):
            toff = tok_base + ci * _CHTOK
            pltpu.sync_copy(idx_hbm.at[pl.ds(toff, _CHTOK)], idx_v)
            copies = []
            for g in range(_NG):
                copies.append(pltpu.async_copy(
                    ptab_hbm.at[idx_v.at[pl.ds(g * _G, _G)]],
                    rows_v.at[pl.ds(g * _G, _G)], sem))
            for c in copies:
                c.wait()

            orow0 = (ci % _FLUSH) * _CHP

            def pooled(p, carry2):
                def lanes(c8, carry3):
                    def tok(t, acc):
                        return acc + rows_v[p * _L + t, pl.ds(c8 * 16, 16)]
                    acc = lax.fori_loop(0, _L, tok, jnp.zeros((16,), jnp.float32))
                    out_v[orow0 + p, pl.ds(c8 * 16, 16)] = acc
                    return carry3
                return lax.fori_loop(0, _PROJ // 16, lanes, carry2)

            lax.fori_loop(0, _CHP, pooled, 0)

            @pl.when((ci + 1) % _FLUSH == 0)
            def _flush():
                fidx = ci // _FLUSH
                pltpu.sync_copy(
                    out_v,
                    out_hbm.at[pl.ds(row_base + fidx * _FLUSH * _CHP,
                                     _FLUSH * _CHP)])
            return carry

        lax.fori_loop(0, _NCHUNK, chunk, 0)

    return pool_kernel(idx_all, ptab)


_RM = 1024
_NM = _B // _RM


def _mlp_body(pa_ref, pb_ref, w1a_ref, w1b_ref, b1_ref, w2_ref, b2_ref,
              o_ref):
    h = jnp.maximum(
        jnp.dot(pa_ref[...], w1a_ref[...], preferred_element_type=jnp.float32)
        + jnp.dot(pb_ref[...], w1b_ref[...], preferred_element_type=jnp.float32)
        + b1_ref[...], 0.0)
    o_ref[...] = jnp.sum(h * w2_ref[...], axis=1) + b2_ref[0, 0]


def _mlp(pooled, w1a, w1b, b1_2, w2r, b2_2):
    return pl.pallas_call(
        _mlp_body,
        grid=(_NM,),
        in_specs=[
            pl.BlockSpec((_RM, _PROJ), lambda i: (i, 0)),
            pl.BlockSpec((_RM, _PROJ), lambda i: (_NM + i, 0)),
            pl.BlockSpec((_PROJ, _PROJ), lambda i: (0, 0)),
            pl.BlockSpec((_PROJ, _PROJ), lambda i: (0, 0)),
            pl.BlockSpec((1, _PROJ), lambda i: (0, 0)),
            pl.BlockSpec((1, _PROJ), lambda i: (0, 0)),
            pl.BlockSpec(memory_space=pltpu.SMEM),
        ],
        out_specs=pl.BlockSpec((_RM,), lambda i: (i,)),
        out_shape=jax.ShapeDtypeStruct((_B,), jnp.float32),
    )(pooled, pooled, w1a, w1b, b1_2, w2r, b2_2)


def kernel(a, b, table, W_proj, b_proj, W1, b1, W2, b2):
    inv_l = jnp.float32(1.0 / _L)
    ptab = _project_table(table, W_proj * inv_l,
                          (b_proj * inv_l).reshape(1, _PROJ))
    idx_all = jnp.concatenate([a.reshape(-1), b.reshape(-1)])
    pooled = _sc_pool(idx_all, ptab)
    return _mlp(pooled, W1[:_PROJ], W1[_PROJ:], b1.reshape(1, _PROJ),
                W2.reshape(1, _PROJ), b2.reshape(1, 1))


# speedup vs baseline: 2.6394x; 2.6394x over previous
import functools

import jax
import jax.numpy as jnp
from jax import lax
from jax.experimental import pallas as pl
from jax.experimental.pallas import tpu as pltpu
from jax.experimental.pallas import tpu_sc as plsc

_VOCAB = 1000000
_EMB = 64
_B = 16384
_L = 50
_PROJ = 128

_RV = 8000
_NV = _VOCAB // _RV


def _proj_body(t_ref, wp_ref, bp_ref, o_ref):
    o_ref[...] = jnp.maximum(
        jnp.dot(t_ref[...], wp_ref[...], preferred_element_type=jnp.float32)
        + bp_ref[...], 0.0)


def _project_table(table, wp_s, bp_s):
    return pl.pallas_call(
        _proj_body,
        grid=(_NV,),
        in_specs=[
            pl.BlockSpec((_RV, _EMB), lambda i: (i, 0)),
            pl.BlockSpec((_EMB, _PROJ), lambda i: (0, 0)),
            pl.BlockSpec((1, _PROJ), lambda i: (0, 0)),
        ],
        out_specs=pl.BlockSpec((_RV, _PROJ), lambda i: (i, 0)),
        out_shape=jax.ShapeDtypeStruct((_VOCAB, _PROJ), jnp.float32),
    )(table, wp_s, bp_s)


_NC = 2
_NS = 16
_NW = _NC * _NS
_NPOOL = 2 * _B
_POOL_PER_W = _NPOOL // _NW
_CHP = 8
_CHTOK = _CHP * _L
_G = 80
_NG = _CHTOK // _G
_NCHUNK = _POOL_PER_W // _CHP
_FLUSH = 8


def _sc_pool(idx_all, ptab):
    mesh = plsc.VectorSubcoreMesh(core_axis_name="c", subcore_axis_name="s")

    @functools.partial(
        pl.kernel,
        mesh=mesh,
        out_type=jax.ShapeDtypeStruct((_NPOOL, _PROJ), jnp.float32),
        scratch_types=[
            pltpu.VMEM((_CHTOK,), jnp.int32),
            pltpu.VMEM((_CHTOK,), jnp.int32),
            pltpu.VMEM((_CHTOK, _PROJ), jnp.float32),
            pltpu.VMEM((_CHTOK, _PROJ), jnp.float32),
            pltpu.VMEM((_FLUSH * _CHP, _PROJ), jnp.float32),
            pltpu.SemaphoreType.DMA,
            pltpu.SemaphoreType.DMA,
            pltpu.SemaphoreType.DMA,
            pltpu.SemaphoreType.DMA,
        ],
    )
    def pool_kernel(idx_hbm, ptab_hbm, out_hbm, idx_v0, idx_v1,
                    rows_v0, rows_v1, out_v, sg0, sg1, si0, si1):
        idx_v = (idx_v0, idx_v1)
        rows_v = (rows_v0, rows_v1)
        sg = (sg0, sg1)
        si = (si0, si1)
        wid = lax.axis_index("s") * _NC + lax.axis_index("c")
        tok_base = wid * _POOL_PER_W * _L
        row_base = wid * _POOL_PER_W

        def idx_copy(c, bu, sem):
            return pltpu.make_async_copy(
                idx_hbm.at[pl.ds(tok_base + c * _CHTOK, _CHTOK)],
                idx_v[bu], sem)

        def gather(g, bu, sem):
            return pltpu.make_async_copy(
                ptab_hbm.at[idx_v[bu].at[pl.ds(g * _G, _G)]],
                rows_v[bu].at[pl.ds(g * _G, _G)], sem)

        def fire_gathers(bu):
            for g in range(_NG):
                gather(g, bu, sg[bu]).start()

        def wait_gathers(bu):
            for g in range(_NG):
                gather(g, bu, sg[bu]).wait()

        def sum_chunk(bu, orow0):
            rows_ref = rows_v[bu]

            def pooled(p, carry2):
                def tok(t2, accs):
                    r = p * _L + 2 * t2
                    a1 = tuple(accs[c] + rows_ref[r, pl.ds(c * 16, 16)]
                               for c in range(8))
                    return tuple(a1[c] + rows_ref[r + 1, pl.ds(c * 16, 16)]
                                 for c in range(8))

                accs = lax.fori_loop(
                    0, _L // 2, tok,
                    tuple(jnp.zeros((16,), jnp.float32) for _ in range(8)))
                for c in range(8):
                    out_v[orow0 + p, pl.ds(c * 16, 16)] = accs[c]
                return carry2

            lax.fori_loop(0, _CHP, pooled, 0)

        def phase(bu, ci):
            nxt = 1 - bu

            @pl.when(ci + 1 < _NCHUNK)
            def _prefetch():
                idx_copy(ci + 1, nxt, si[nxt]).wait()
                fire_gathers(nxt)

            wait_gathers(bu)

            @pl.when(ci + 2 < _NCHUNK)
            def _next_idx():
                idx_copy(ci + 2, bu, si[bu]).start()

            sum_chunk(bu, (ci % _FLUSH) * _CHP)

            @pl.when((ci + 1) % _FLUSH == 0)
            def _flush():
                fidx = ci // _FLUSH
                pltpu.sync_copy(
                    out_v,
                    out_hbm.at[pl.ds(row_base + fidx * _FLUSH * _CHP,
                                     _FLUSH * _CHP)])

        pltpu.sync_copy(idx_hbm.at[pl.ds(tok_base, _CHTOK)], idx_v[0])
        fire_gathers(0)
        idx_copy(1, 1, si[1]).start()

        def pair(k, carry):
            phase(0, 2 * k)
            phase(1, 2 * k + 1)
            return carry

        lax.fori_loop(0, _NCHUNK // 2, pair, 0)

    return pool_kernel(idx_all, ptab)


_RM = 1024
_NM = _B // _RM


def _mlp_body(pa_ref, pb_ref, w1a_ref, w1b_ref, b1_ref, w2_ref, b2_ref,
              o_ref):
    h = jnp.maximum(
        jnp.dot(pa_ref[...], w1a_ref[...], preferred_element_type=jnp.float32)
        + jnp.dot(pb_ref[...], w1b_ref[...], preferred_element_type=jnp.float32)
        + b1_ref[...], 0.0)
    o_ref[...] = jnp.sum(h * w2_ref[...], axis=1) + b2_ref[0, 0]


def _mlp(pooled, w1a, w1b, b1_2, w2r, b2_2):
    return pl.pallas_call(
        _mlp_body,
        grid=(_NM,),
        in_specs=[
            pl.BlockSpec((_RM, _PROJ), lambda i: (i, 0)),
            pl.BlockSpec((_RM, _PROJ), lambda i: (_NM + i, 0)),
            pl.BlockSpec((_PROJ, _PROJ), lambda i: (0, 0)),
            pl.BlockSpec((_PROJ, _PROJ), lambda i: (0, 0)),
            pl.BlockSpec((1, _PROJ), lambda i: (0, 0)),
            pl.BlockSpec((1, _PROJ), lambda i: (0, 0)),
            pl.BlockSpec(memory_space=pltpu.SMEM),
        ],
        out_specs=pl.BlockSpec((_RM,), lambda i: (i,)),
        out_shape=jax.ShapeDtypeStruct((_B,), jnp.float32),
    )(pooled, pooled, w1a, w1b, b1_2, w2r, b2_2)


def kernel(a, b, table, W_proj, b_proj, W1, b1, W2, b2):
    inv_l = jnp.float32(1.0 / _L)
    ptab = _project_table(table, W_proj * inv_l,
                          (b_proj * inv_l).reshape(1, _PROJ))
    idx_all = jnp.concatenate([a.reshape(-1), b.reshape(-1)])
    pooled = _sc_pool(idx_all, ptab)
    return _mlp(pooled, W1[:_PROJ], W1[_PROJ:], b1.reshape(1, _PROJ),
                W2.reshape(1, _PROJ), b2.reshape(1, 1))


# speedup vs baseline: 2.6701x; 1.0116x over previous
import functools

import jax
import jax.numpy as jnp
from jax import lax
from jax.experimental import pallas as pl
from jax.experimental.pallas import tpu as pltpu
from jax.experimental.pallas import tpu_sc as plsc

_VOCAB = 1000000
_EMB = 64
_B = 16384
_L = 50
_PROJ = 128

_RV = 8000
_NV = _VOCAB // _RV


def _proj_body(t_ref, wp_ref, bp_ref, o_ref):
    o_ref[...] = jnp.maximum(
        jnp.dot(t_ref[...], wp_ref[...], preferred_element_type=jnp.float32)
        + bp_ref[...], 0.0)


def _project_table(table, wp_s, bp_s):
    return pl.pallas_call(
        _proj_body,
        grid=(_NV,),
        in_specs=[
            pl.BlockSpec((_RV, _EMB), lambda i: (i, 0)),
            pl.BlockSpec((_EMB, _PROJ), lambda i: (0, 0)),
            pl.BlockSpec((1, _PROJ), lambda i: (0, 0)),
        ],
        out_specs=pl.BlockSpec((_RV, _PROJ), lambda i: (i, 0)),
        out_shape=jax.ShapeDtypeStruct((_VOCAB, _PROJ), jnp.float32),
    )(table, wp_s, bp_s)


_NC = 2
_NS = 16
_NW = _NC * _NS
_NPOOL = 2 * _B
_POOL_PER_W = _NPOOL // _NW
_CHP = 8
_SLOT = 56
_NCHUNK = _POOL_PER_W // _CHP
_FLUSH = 8


def _sc_pool(idx2, ptab):
    mesh = plsc.VectorSubcoreMesh(core_axis_name="c", subcore_axis_name="s")

    @functools.partial(
        pl.kernel,
        mesh=mesh,
        out_type=jax.ShapeDtypeStruct((_NPOOL, _PROJ), jnp.float32),
        scratch_types=[
            pltpu.VMEM((_CHP, _SLOT), jnp.int32),
            pltpu.VMEM((_CHP, _SLOT), jnp.int32),
            pltpu.VMEM((_CHP * _SLOT, _PROJ), jnp.float32),
            pltpu.VMEM((_CHP * _SLOT, _PROJ), jnp.float32),
            pltpu.VMEM((_FLUSH * _CHP, _PROJ), jnp.float32),
            pltpu.SemaphoreType.DMA,
            pltpu.SemaphoreType.DMA,
            pltpu.SemaphoreType.DMA,
            pltpu.SemaphoreType.DMA,
        ],
    )
    def pool_kernel(idx_hbm, ptab_hbm, out_hbm, idx_v0, idx_v1,
                    rows_v0, rows_v1, out_v, sg0, sg1, si0, si1):
        idx_v = (idx_v0, idx_v1)
        rows_v = (rows_v0, rows_v1)
        sg = (sg0, sg1)
        si = (si0, si1)
        wid = lax.axis_index("s") * _NC + lax.axis_index("c")
        row_base = wid * _POOL_PER_W

        def idx_copy(c, bu, sem):
            return pltpu.make_async_copy(
                idx_hbm.at[pl.ds(row_base + c * _CHP, _CHP), :],
                idx_v[bu], sem)

        def gather(p, bu, sem):
            return pltpu.make_async_copy(
                ptab_hbm.at[idx_v[bu].at[p].at[pl.ds(0, _L)]],
                rows_v[bu].at[pl.ds(p * _SLOT, _L)], sem)

        def fire_gathers(bu):
            for p in range(_CHP):
                gather(p, bu, sg[bu]).start()

        def wait_gathers(bu):
            for p in range(_CHP):
                gather(p, bu, sg[bu]).wait()

        def sum_chunk(bu, orow0):
            rows_ref = rows_v[bu]

            def pooled(p, carry2):
                def tok(t2, accs):
                    r = p * _SLOT + 2 * t2
                    a1 = tuple(accs[c] + rows_ref[r, pl.ds(c * 16, 16)]
                               for c in range(8))
                    return tuple(a1[c] + rows_ref[r + 1, pl.ds(c * 16, 16)]
                                 for c in range(8))

                accs = lax.fori_loop(
                    0, _L // 2, tok,
                    tuple(jnp.zeros((16,), jnp.float32) for _ in range(8)))
                for c in range(8):
                    out_v[orow0 + p, pl.ds(c * 16, 16)] = accs[c]
                return carry2

            lax.fori_loop(0, _CHP, pooled, 0)

        def phase(bu, ci):
            nxt = 1 - bu

            @pl.when(ci + 1 < _NCHUNK)
            def _prefetch():
                idx_copy(ci + 1, nxt, si[nxt]).wait()
                fire_gathers(nxt)

            wait_gathers(bu)

            @pl.when(ci + 2 < _NCHUNK)
            def _next_idx():
                idx_copy(ci + 2, bu, si[bu]).start()

            sum_chunk(bu, (ci % _FLUSH) * _CHP)

            @pl.when((ci + 1) % _FLUSH == 0)
            def _flush():
                fidx = ci // _FLUSH
                pltpu.sync_copy(
                    out_v,
                    out_hbm.at[pl.ds(row_base + fidx * _FLUSH * _CHP,
                                     _FLUSH * _CHP)])

        pltpu.sync_copy(idx_hbm.at[pl.ds(row_base, _CHP), :], idx_v[0])
        fire_gathers(0)
        idx_copy(1, 1, si[1]).start()

        def pair(k, carry):
            phase(0, 2 * k)
            phase(1, 2 * k + 1)
            return carry

        lax.fori_loop(0, _NCHUNK // 2, pair, 0)

    return pool_kernel(idx2, ptab)


_RM = 1024
_NM = _B // _RM


def _mlp_body(pa_ref, pb_ref, w1a_ref, w1b_ref, b1_ref, w2_ref, b2_ref,
              o_ref):
    h = jnp.maximum(
        jnp.dot(pa_ref[...], w1a_ref[...], preferred_element_type=jnp.float32)
        + jnp.dot(pb_ref[...], w1b_ref[...], preferred_element_type=jnp.float32)
        + b1_ref[...], 0.0)
    o_ref[...] = jnp.sum(h * w2_ref[...], axis=1) + b2_ref[0, 0]


def _mlp(pooled, w1a, w1b, b1_2, w2r, b2_2):
    return pl.pallas_call(
        _mlp_body,
        grid=(_NM,),
        in_specs=[
            pl.BlockSpec((_RM, _PROJ), lambda i: (i, 0)),
            pl.BlockSpec((_RM, _PROJ), lambda i: (_NM + i, 0)),
            pl.BlockSpec((_PROJ, _PROJ), lambda i: (0, 0)),
            pl.BlockSpec((_PROJ, _PROJ), lambda i: (0, 0)),
            pl.BlockSpec((1, _PROJ), lambda i: (0, 0)),
            pl.BlockSpec((1, _PROJ), lambda i: (0, 0)),
            pl.BlockSpec(memory_space=pltpu.SMEM),
        ],
        out_specs=pl.BlockSpec((_RM,), lambda i: (i,)),
        out_shape=jax.ShapeDtypeStruct((_B,), jnp.float32),
    )(pooled, pooled, w1a, w1b, b1_2, w2r, b2_2)


def kernel(a, b, table, W_proj, b_proj, W1, b1, W2, b2):
    inv_l = jnp.float32(1.0 / _L)
    ptab = _project_table(table, W_proj * inv_l,
                          (b_proj * inv_l).reshape(1, _PROJ))
    idx2 = jnp.pad(jnp.concatenate([a, b], axis=0), ((0, 0), (0, _SLOT - _L)))
    pooled = _sc_pool(idx2, ptab)
    return _mlp(pooled, W1[:_PROJ], W1[_PROJ:], b1.reshape(1, _PROJ),
                W2.reshape(1, _PROJ), b2.reshape(1, 1))
